# Initial kernel scaffold; baseline (speedup 1.0000x reference)
import jax
import jax.numpy as jnp
from jax.experimental import pallas as pl


def kernel(x, topk_idx, weights, W0, b0, W1, b1):
    raise NotImplementedError("write your pallas kernel here")


# speedup vs baseline: 4.1774x; 4.1774x over previous
import functools

import jax
import jax.numpy as jnp
from jax.experimental import pallas as pl
from jax.experimental.pallas import tpu as pltpu

_BLK = 256


def _fused_body(t0_ref, t1_ref, w_ref, x_ref, W0_ref, b0_ref, W1_ref, b1_ref,
                out_ref, carry_ref, *, blk, d, out_dim):
    i = pl.program_id(0)
    xb = x_ref[...]
    t0 = t0_ref[...]
    t1 = t1_ref[...]
    m0 = (t0 == 0) | (t1 == 0)
    m1 = (t0 == 1) | (t1 == 1)
    o0 = jnp.where(
        m0,
        jnp.dot(xb, W0_ref[...], preferred_element_type=jnp.float32)
        + b0_ref[...],
        0.0,
    )

    @pl.when(i == 0)
    def _():
        carry_ref[...] = o0[0:1, :]

    f = o0
    v = m0.astype(jnp.int32)
    s = 1
    while s < blk:
        f = jnp.where(v > 0, f, jnp.concatenate([f[:s], f[:-s]], axis=0))
        v = jnp.maximum(v, jnp.concatenate([v[:s], v[:-s]], axis=0))
        s *= 2
    filled = jnp.where(v > 0, f, carry_ref[...])
    carry_ref[...] = filled[blk - 1:blk, :]

    o1 = jnp.where(
        m1,
        jnp.dot(xb, W1_ref[0:d, :], preferred_element_type=jnp.float32)
        + jnp.dot(filled, W1_ref[d:, :], preferred_element_type=jnp.float32)
        + b1_ref[...],
        0.0,
    )
    w = w_ref[...]
    out_ref[...] = w * o0 + (1.0 - w) * o1


def _fused_dense(t0, t1, w0, x, W0, b0, W1, b1):
    T, D = x.shape
    OUT = W0.shape[1]
    blk = _BLK
    grid = (T // blk,)
    body = functools.partial(_fused_body, blk=blk, d=D, out_dim=OUT)
    return pl.pallas_call(
        body,
        grid=grid,
        in_specs=[
            pl.BlockSpec((blk, 1), lambda i: (i, 0)),
            pl.BlockSpec((blk, 1), lambda i: (i, 0)),
            pl.BlockSpec((blk, 1), lambda i: (i, 0)),
            pl.BlockSpec((blk, D), lambda i: (i, 0)),
            pl.BlockSpec((D, OUT), lambda i: (0, 0)),
            pl.BlockSpec((1, OUT), lambda i: (0, 0)),
            pl.BlockSpec((D + OUT, OUT), lambda i: (0, 0)),
            pl.BlockSpec((1, OUT), lambda i: (0, 0)),
        ],
        out_specs=pl.BlockSpec((blk, OUT), lambda i: (i, 0)),
        out_shape=jax.ShapeDtypeStruct((T, OUT), jnp.float32),
        scratch_shapes=[pltpu.VMEM((1, OUT), jnp.float32)],
        compiler_params=pltpu.CompilerParams(
            dimension_semantics=("arbitrary",),
        ),
    )(t0, t1, w0, x, W0, b0, W1, b1)


def kernel(x, topk_idx, weights, W0, b0, W1, b1):
    t0 = topk_idx[:, 0:1].astype(jnp.int32)
    t1 = topk_idx[:, 1:2].astype(jnp.int32)
    w0 = weights[:, 0:1]
    return _fused_dense(t0, t1, w0, x, W0,
                        b0.reshape(1, -1), W1, b1.reshape(1, -1))
